# baseline (speedup 1.0000x reference)
import functools

import jax
import jax.numpy as jnp
from jax import lax
from jax.experimental import pallas as pl
from jax.experimental.pallas import tpu as pltpu
from jax.experimental.pallas import tpu_sc as plsc

N_NODES = 50000
NB = 200
N_CHUNKS = N_NODES // NB
EB_ROWS = 2
EB = EB_ROWS * 128
N_SINK = N_NODES + 8


def _mesh():
    return plsc.VectorSubcoreMesh(core_axis_name="c", subcore_axis_name="s")


def _deg_body(n_echunks, dst2, out, acc, dst_v, ones_v, zbuf):
    cid = lax.axis_index("c")
    sid = lax.axis_index("s")
    wid = cid * 16 + sid

    one16 = jnp.full((16,), 1.0, jnp.float32)
    zero16 = jnp.zeros((16,), jnp.float32)
    for i in range(8):
        ones_v[pl.ds(i * 16, 16)] = one16

    def zfill(i, carry):
        zbuf[pl.ds(i * 16, 16)] = zero16
        return carry
    lax.fori_loop(0, 63, zfill, 0)

    def zchunk(j, carry):
        c = sid + 16 * j
        @pl.when(c < N_CHUNKS)
        def _():
            pltpu.sync_copy(zbuf.at[pl.ds(0, NB)], acc.at[pl.ds(c * NB, NB)])
        return carry
    lax.fori_loop(0, (N_CHUNKS + 15) // 16, zchunk, 0)
    plsc.subcore_barrier()

    n_iter = (n_echunks + 31) // 32

    def echunk(j, carry):
        m = wid + 32 * j
        @pl.when(m < n_echunks)
        def _():
            pltpu.sync_copy(dst2.at[m], dst_v)
            for r in range(EB_ROWS):
                pltpu.sync_copy(ones_v, acc.at[dst_v.at[r]], add=True)
        return carry
    lax.fori_loop(0, n_iter, echunk, 0)
    plsc.subcore_barrier()

    def wchunk(j, carry):
        c = sid + 16 * j
        @pl.when(c < N_CHUNKS)
        def _():
            pltpu.sync_copy(acc.at[pl.ds(c * NB, NB)], zbuf.at[pl.ds(0, NB)])
            pltpu.sync_copy(zbuf.at[pl.ds(0, NB)],
                            out.at[pl.ds(cid * N_NODES + c * NB, NB)])
        return carry
    lax.fori_loop(0, (N_CHUNKS + 15) // 16, wchunk, 0)


def _deg_call(dst2):
    n_echunks = dst2.shape[0]
    kern = functools.partial(
        pl.kernel,
        mesh=_mesh(),
        compiler_params=pltpu.CompilerParams(use_tc_tiling_on_sc=False),
        out_type=jax.ShapeDtypeStruct((2 * N_NODES,), jnp.float32),
        scratch_types=[
            pltpu.VMEM_SHARED((N_SINK,), jnp.float32),
            pltpu.VMEM((EB_ROWS, 128), jnp.int32),
            pltpu.VMEM((128,), jnp.float32),
            pltpu.VMEM((1008,), jnp.float32),
        ],
    )(functools.partial(_deg_body, n_echunks))
    return kern(dst2)


def _mlp_body(x_ref, w1_ref, b1_ref, w2_ref, b2_ref, deg_ref, t0_ref,
              hid_ref, yk_ref, dinv_ref):
    xb = x_ref[...]
    h1 = jnp.maximum(
        jnp.dot(xb, w1_ref[...], preferred_element_type=jnp.float32)
        + b1_ref[...], 0.0)
    h = jnp.maximum(
        jnp.dot(h1, w2_ref[...], preferred_element_type=jnp.float32)
        + b2_ref[...], 0.0)
    dinv = lax.rsqrt(deg_ref[...])
    hid = t0_ref[...] * h
    y = dinv * h
    hid_ref[0, :, :] = hid[:, :32]
    hid_ref[1, :, :] = hid[:, 32:]
    yk_ref[0, :, :] = y[:, :32]
    yk_ref[1, :, :] = y[:, 32:]
    dinv_ref[...] = dinv


def _mlp_call(x, W1, b1, W2, b2, deg, t0):
    n, nfeat = x.shape
    nhid = W1.shape[1]
    ncls = W2.shape[1]
    B = 1000
    grid = (n // B,)
    return pl.pallas_call(
        _mlp_body,
        grid=grid,
        in_specs=[
            pl.BlockSpec((B, nfeat), lambda i: (i, 0)),
            pl.BlockSpec((nfeat, nhid), lambda i: (0, 0)),
            pl.BlockSpec((1, nhid), lambda i: (0, 0)),
            pl.BlockSpec((nhid, ncls), lambda i: (0, 0)),
            pl.BlockSpec((1, ncls), lambda i: (0, 0)),
            pl.BlockSpec((B, 1), lambda i: (i, 0)),
            pl.BlockSpec((1, 1), lambda i: (0, 0)),
        ],
        out_specs=[
            pl.BlockSpec((2, B, 32), lambda i: (0, i, 0)),
            pl.BlockSpec((2, B, 32), lambda i: (0, i, 0)),
            pl.BlockSpec((B, 1), lambda i: (i, 0)),
        ],
        out_shape=[
            jax.ShapeDtypeStruct((2, n, 32), jnp.float32),
            jax.ShapeDtypeStruct((2, n, 32), jnp.float32),
            jax.ShapeDtypeStruct((n, 1), jnp.float32),
        ],
    )(x, W1, b1.reshape(1, nhid), W2, b2.reshape(1, ncls), deg, t0)


def _hop_body(n_echunks, yk, hid_in, srcA, srcB, dst2, dinv, tk,
              yk_out, hid_out,
              acc, src_vA, src_vB, dst_vA, dst_vB, rows_vA, rows_vB,
              nbuf_a, nbuf_h, dinv_v, tk_v,
              gsemA, gsemB, ssemA, ssemB):
    cid = lax.axis_index("c")
    sid = lax.axis_index("s")
    half = cid * N_NODES

    def p1(j, carry):
        c = sid + 16 * j
        @pl.when(c < N_CHUNKS)
        def _():
            pltpu.sync_copy(yk.at[pl.ds(half + c * NB, NB)], nbuf_a)
            pltpu.sync_copy(nbuf_a, acc.at[pl.ds(c * NB, NB)])
        return carry
    lax.fori_loop(0, (N_CHUNKS + 15) // 16, p1, 0)
    plsc.subcore_barrier()

    n_pairs = (n_echunks + 31) // 32

    def _idx(m, src_v, dst_v):
        @pl.when(cid == 0)
        def _():
            pltpu.sync_copy(srcA.at[m], src_v)
        @pl.when(cid == 1)
        def _():
            pltpu.sync_copy(srcB.at[m], src_v)
        pltpu.sync_copy(dst2.at[m], dst_v)

    def _fire_g(src_v, rows_v, gsem):
        for r in range(EB_ROWS):
            pltpu.async_copy(yk.at[src_v.at[r]],
                             rows_v.at[pl.ds(r * 128, 128)], gsem)

    def _wait_g(src_v, rows_v, gsem):
        for r in range(EB_ROWS):
            pltpu.make_async_copy(yk.at[src_v.at[r]],
                                  rows_v.at[pl.ds(r * 128, 128)],
                                  gsem).wait()

    def _fire_s(dst_v, rows_v, ssem):
        for r in range(EB_ROWS):
            pltpu.async_copy(rows_v.at[pl.ds(r * 128, 128)],
                             acc.at[dst_v.at[r]], ssem, add=True)

    def _wait_s(dst_v, rows_v, ssem):
        for r in range(EB_ROWS):
            pltpu.make_async_copy(rows_v.at[pl.ds(r * 128, 128)],
                                  acc.at[dst_v.at[r]], ssem).wait()

    def p2(j, carry):
        mA = sid + 16 * (2 * j)
        mB = sid + 16 * (2 * j + 1)
        wA = mA < n_echunks
        wB = mB < n_echunks

        @pl.when(wA)
        def _():
            _idx(mA, src_vA, dst_vA)
            _fire_g(src_vA, rows_vA, gsemA)

        @pl.when(wB)
        def _():
            _idx(mB, src_vB, dst_vB)

        @pl.when(wA)
        def _():
            _wait_g(src_vA, rows_vA, gsemA)
            _fire_s(dst_vA, rows_vA, ssemA)

        @pl.when(wB)
        def _():
            _fire_g(src_vB, rows_vB, gsemB)

        @pl.when(wB)
        def _():
            _wait_g(src_vB, rows_vB, gsemB)
            _fire_s(dst_vB, rows_vB, ssemB)

        @pl.when(wA)
        def _():
            _wait_s(dst_vA, rows_vA, ssemA)

        @pl.when(wB)
        def _():
            _wait_s(dst_vB, rows_vB, ssemB)
        return carry
    lax.fori_loop(0, n_pairs, p2, 0)
    plsc.subcore_barrier()

    pltpu.sync_copy(tk, tk_v)
    tkv = tk_v[...]

    def p3(j, carry):
        c = sid + 16 * j
        @pl.when(c < N_CHUNKS)
        def _():
            b = c * NB
            pltpu.sync_copy(acc.at[pl.ds(b, NB)], nbuf_a)
            pltpu.sync_copy(hid_in.at[pl.ds(half + b, NB)], nbuf_h)
            pltpu.sync_copy(dinv.at[pl.ds(b, NB)], dinv_v.at[pl.ds(0, NB)])

            def row(i, carry2):
                dsp = dinv_v[pl.ds(i, 16)][0]
                a0 = nbuf_a[i, pl.ds(0, 16)]
                a1 = nbuf_a[i, pl.ds(16, 16)]
                x0 = a0 * dsp
                x1 = a1 * dsp
                nbuf_h[i, pl.ds(0, 16)] = nbuf_h[i, pl.ds(0, 16)] + tkv * x0
                nbuf_h[i, pl.ds(16, 16)] = nbuf_h[i, pl.ds(16, 16)] + tkv * x1
                nbuf_a[i, pl.ds(0, 16)] = x0 * dsp
                nbuf_a[i, pl.ds(16, 16)] = x1 * dsp
                return carry2
            lax.fori_loop(0, NB, row, 0)
            pltpu.sync_copy(nbuf_h, hid_out.at[pl.ds(half + b, NB)])
            pltpu.sync_copy(nbuf_a, yk_out.at[pl.ds(half + b, NB)])
        return carry
    lax.fori_loop(0, (N_CHUNKS + 15) // 16, p3, 0)


def _hop_call(yk, hid, srcA, srcB, dst2, dinv, tk):
    n_echunks = dst2.shape[0]
    kern = functools.partial(
        pl.kernel,
        mesh=_mesh(),
        compiler_params=pltpu.CompilerParams(use_tc_tiling_on_sc=False),
        out_type=[
            jax.ShapeDtypeStruct((2 * N_NODES, 32), jnp.float32),
            jax.ShapeDtypeStruct((2 * N_NODES, 32), jnp.float32),
        ],
        scratch_types=[
            pltpu.VMEM_SHARED((N_SINK, 32), jnp.float32),
            pltpu.VMEM((EB_ROWS, 128), jnp.int32),
            pltpu.VMEM((EB_ROWS, 128), jnp.int32),
            pltpu.VMEM((EB_ROWS, 128), jnp.int32),
            pltpu.VMEM((EB_ROWS, 128), jnp.int32),
            pltpu.VMEM((EB, 32), jnp.float32),
            pltpu.VMEM((EB, 32), jnp.float32),
            pltpu.VMEM((NB, 32), jnp.float32),
            pltpu.VMEM((NB, 32), jnp.float32),
            pltpu.VMEM((NB + 16,), jnp.float32),
            pltpu.VMEM((16,), jnp.float32),
            pltpu.SemaphoreType.DMA,
            pltpu.SemaphoreType.DMA,
            pltpu.SemaphoreType.DMA,
            pltpu.SemaphoreType.DMA,
        ],
    )(functools.partial(_hop_body, n_echunks))
    return kern(yk, hid, srcA, srcB, dst2, dinv, tk)


def kernel(x, edge_index, W1, b1, W2, b2, temp):
    n = x.shape[0]
    e = edge_index.shape[1]
    assert n == N_NODES
    src = edge_index[0]
    dst = edge_index[1]
    e_pad = ((e + EB - 1) // EB) * EB
    if e_pad != e:
        pad = e_pad - e
        src = jnp.concatenate([src, jnp.zeros((pad,), src.dtype)])
        dst = jnp.concatenate(
            [dst, jnp.full((pad,), N_NODES, dst.dtype)])
    src2A = src.reshape(e_pad // EB, EB_ROWS, 128)
    src2B = (src + n).reshape(e_pad // EB, EB_ROWS, 128)
    dst2 = dst.reshape(e_pad // EB, EB_ROWS, 128)

    partials = _deg_call(dst2)
    deg = partials[:n] + partials[n:] + 1.0

    hid, yk, dinv = _mlp_call(x, W1, b1, W2, b2, deg.reshape(n, 1),
                              temp[0].reshape(1, 1))
    hid = hid.reshape(2 * n, 32)
    yk = yk.reshape(2 * n, 32)
    dinv = dinv.reshape(n)

    k_hops = temp.shape[0] - 1
    for k in range(k_hops):
        tk = jnp.broadcast_to(temp[k + 1], (16,))
        yk, hid = _hop_call(yk, hid, src2A, src2B, dst2, dinv, tk)

    hid = hid.reshape(2, n, 32)
    return jnp.concatenate([hid[0], hid[1]], axis=1)


# speedup vs baseline: 1.4478x; 1.4478x over previous
import functools

import jax
import jax.numpy as jnp
from jax import lax
from jax.experimental import pallas as pl
from jax.experimental.pallas import tpu as pltpu
from jax.experimental.pallas import tpu_sc as plsc

N_NODES = 50000
NB = 200
N_CHUNKS = N_NODES // NB
EB_ROWS = 16
EB = EB_ROWS * 128
N_STAGES = EB_ROWS // 2
N_SINK = N_NODES + 8


def _mesh():
    return plsc.VectorSubcoreMesh(core_axis_name="c", subcore_axis_name="s")


def _deg_body(n_echunks, dst2, out, acc, dst_v, ones_v, zbuf):
    cid = lax.axis_index("c")
    sid = lax.axis_index("s")
    wid = cid * 16 + sid

    one16 = jnp.full((16,), 1.0, jnp.float32)
    zero16 = jnp.zeros((16,), jnp.float32)
    for i in range(8):
        ones_v[pl.ds(i * 16, 16)] = one16

    def zfill(i, carry):
        zbuf[pl.ds(i * 16, 16)] = zero16
        return carry
    lax.fori_loop(0, 63, zfill, 0)

    def zchunk(j, carry):
        c = sid + 16 * j
        @pl.when(c < N_CHUNKS)
        def _():
            pltpu.sync_copy(zbuf.at[pl.ds(0, NB)], acc.at[pl.ds(c * NB, NB)])
        return carry
    lax.fori_loop(0, (N_CHUNKS + 15) // 16, zchunk, 0)
    plsc.subcore_barrier()

    n_iter = (n_echunks + 31) // 32

    def echunk(j, carry):
        m = wid + 32 * j
        @pl.when(m < n_echunks)
        def _():
            pltpu.sync_copy(dst2.at[m], dst_v)
            for r in range(EB_ROWS):
                pltpu.sync_copy(ones_v, acc.at[dst_v.at[r]], add=True)
        return carry
    lax.fori_loop(0, n_iter, echunk, 0)
    plsc.subcore_barrier()

    def wchunk(j, carry):
        c = sid + 16 * j
        @pl.when(c < N_CHUNKS)
        def _():
            pltpu.sync_copy(acc.at[pl.ds(c * NB, NB)], zbuf.at[pl.ds(0, NB)])
            pltpu.sync_copy(zbuf.at[pl.ds(0, NB)],
                            out.at[pl.ds(cid * N_NODES + c * NB, NB)])
        return carry
    lax.fori_loop(0, (N_CHUNKS + 15) // 16, wchunk, 0)


def _deg_call(dst2):
    n_echunks = dst2.shape[0]
    kern = functools.partial(
        pl.kernel,
        mesh=_mesh(),
        compiler_params=pltpu.CompilerParams(use_tc_tiling_on_sc=False),
        out_type=jax.ShapeDtypeStruct((2 * N_NODES,), jnp.float32),
        scratch_types=[
            pltpu.VMEM_SHARED((N_SINK,), jnp.float32),
            pltpu.VMEM((EB_ROWS, 128), jnp.int32),
            pltpu.VMEM((128,), jnp.float32),
            pltpu.VMEM((1008,), jnp.float32),
        ],
    )(functools.partial(_deg_body, n_echunks))
    return kern(dst2)


def _mlp_body(x_ref, w1_ref, b1_ref, w2_ref, b2_ref, deg_ref, t0_ref,
              hid_ref, yk_ref, dinv_ref):
    xb = x_ref[...]
    h1 = jnp.maximum(
        jnp.dot(xb, w1_ref[...], preferred_element_type=jnp.float32)
        + b1_ref[...], 0.0)
    h = jnp.maximum(
        jnp.dot(h1, w2_ref[...], preferred_element_type=jnp.float32)
        + b2_ref[...], 0.0)
    dinv = lax.rsqrt(deg_ref[...])
    hid = t0_ref[...] * h
    y = dinv * h
    hid_ref[0, :, :] = hid[:, :32]
    hid_ref[1, :, :] = hid[:, 32:]
    yk_ref[0, :, :] = y[:, :32]
    yk_ref[1, :, :] = y[:, 32:]
    dinv_ref[...] = dinv


def _mlp_call(x, W1, b1, W2, b2, deg, t0):
    n, nfeat = x.shape
    nhid = W1.shape[1]
    ncls = W2.shape[1]
    B = 1000
    grid = (n // B,)
    return pl.pallas_call(
        _mlp_body,
        grid=grid,
        in_specs=[
            pl.BlockSpec((B, nfeat), lambda i: (i, 0)),
            pl.BlockSpec((nfeat, nhid), lambda i: (0, 0)),
            pl.BlockSpec((1, nhid), lambda i: (0, 0)),
            pl.BlockSpec((nhid, ncls), lambda i: (0, 0)),
            pl.BlockSpec((1, ncls), lambda i: (0, 0)),
            pl.BlockSpec((B, 1), lambda i: (i, 0)),
            pl.BlockSpec((1, 1), lambda i: (0, 0)),
        ],
        out_specs=[
            pl.BlockSpec((2, B, 32), lambda i: (0, i, 0)),
            pl.BlockSpec((2, B, 32), lambda i: (0, i, 0)),
            pl.BlockSpec((B, 1), lambda i: (i, 0)),
        ],
        out_shape=[
            jax.ShapeDtypeStruct((2, n, 32), jnp.float32),
            jax.ShapeDtypeStruct((2, n, 32), jnp.float32),
            jax.ShapeDtypeStruct((n, 1), jnp.float32),
        ],
    )(x, W1, b1.reshape(1, nhid), W2, b2.reshape(1, ncls), deg, t0)


def _hop_body(n_echunks, yk, hid_in, srcA, srcB, dst2, dinv, tk,
              yk_out, hid_out,
              acc, src_v, dst_v, rows_vA, rows_vB, dinv_v, tk_v,
              gsemA, gsemB, ssemA, ssemB):
    cid = lax.axis_index("c")
    sid = lax.axis_index("s")
    half = cid * N_NODES
    nbuf_a = rows_vA.at[pl.ds(0, NB)]
    nbuf_h = rows_vB.at[pl.ds(0, NB)]

    def p1(j, carry):
        c = sid + 16 * j
        @pl.when(c < N_CHUNKS)
        def _():
            pltpu.sync_copy(yk.at[pl.ds(half + c * NB, NB)], nbuf_a)
            pltpu.sync_copy(nbuf_a, acc.at[pl.ds(c * NB, NB)])
        return carry
    lax.fori_loop(0, (N_CHUNKS + 15) // 16, p1, 0)
    plsc.subcore_barrier()

    rows = (rows_vA, rows_vB)
    gsems = (gsemA, gsemB)
    ssems = (ssemA, ssemB)

    def _fire_g(stage):
        sl = stage & 1
        for r in (2 * stage, 2 * stage + 1):
            pltpu.async_copy(yk.at[src_v.at[r]],
                             rows[sl].at[pl.ds((r & 1) * 128, 128)],
                             gsems[sl])

    def _wait_g(stage):
        sl = stage & 1
        for r in (2 * stage, 2 * stage + 1):
            pltpu.make_async_copy(yk.at[src_v.at[r]],
                                  rows[sl].at[pl.ds((r & 1) * 128, 128)],
                                  gsems[sl]).wait()

    def _fire_s(stage):
        sl = stage & 1
        for r in (2 * stage, 2 * stage + 1):
            pltpu.async_copy(rows[sl].at[pl.ds((r & 1) * 128, 128)],
                             acc.at[dst_v.at[r]], ssems[sl], add=True)

    def _wait_s(stage):
        sl = stage & 1
        for r in (2 * stage, 2 * stage + 1):
            pltpu.make_async_copy(rows[sl].at[pl.ds((r & 1) * 128, 128)],
                                  acc.at[dst_v.at[r]], ssems[sl]).wait()

    def p2(j, carry):
        m = sid + 16 * j
        @pl.when(m < n_echunks)
        def _():
            @pl.when(cid == 0)
            def _():
                pltpu.sync_copy(srcA.at[m], src_v)
            @pl.when(cid == 1)
            def _():
                pltpu.sync_copy(srcB.at[m], src_v)
            pltpu.sync_copy(dst2.at[m], dst_v)
            _fire_g(0)
            for st in range(N_STAGES):
                if st + 1 < N_STAGES:
                    if st >= 1:
                        _wait_s(st - 1)
                    _fire_g(st + 1)
                _wait_g(st)
                _fire_s(st)
            _wait_s(N_STAGES - 2)
            _wait_s(N_STAGES - 1)
        return carry
    lax.fori_loop(0, (n_echunks + 15) // 16, p2, 0)
    plsc.subcore_barrier()

    pltpu.sync_copy(tk, tk_v)
    tkv = tk_v[...]

    def p3(j, carry):
        c = sid + 16 * j
        @pl.when(c < N_CHUNKS)
        def _():
            b = c * NB
            pltpu.sync_copy(acc.at[pl.ds(b, NB)], nbuf_a)
            pltpu.sync_copy(hid_in.at[pl.ds(half + b, NB)], nbuf_h)
            pltpu.sync_copy(dinv.at[pl.ds(b, NB)], dinv_v.at[pl.ds(0, NB)])

            def row(i, carry2):
                dsp = dinv_v[pl.ds(i, 16)][0]
                a0 = nbuf_a[i, pl.ds(0, 16)]
                a1 = nbuf_a[i, pl.ds(16, 16)]
                x0 = a0 * dsp
                x1 = a1 * dsp
                nbuf_h[i, pl.ds(0, 16)] = nbuf_h[i, pl.ds(0, 16)] + tkv * x0
                nbuf_h[i, pl.ds(16, 16)] = nbuf_h[i, pl.ds(16, 16)] + tkv * x1
                nbuf_a[i, pl.ds(0, 16)] = x0 * dsp
                nbuf_a[i, pl.ds(16, 16)] = x1 * dsp
                return carry2
            lax.fori_loop(0, NB, row, 0)
            pltpu.sync_copy(nbuf_h, hid_out.at[pl.ds(half + b, NB)])
            pltpu.sync_copy(nbuf_a, yk_out.at[pl.ds(half + b, NB)])
        return carry
    lax.fori_loop(0, (N_CHUNKS + 15) // 16, p3, 0)


def _hop_call(yk, hid, srcA, srcB, dst2, dinv, tk):
    n_echunks = dst2.shape[0]
    kern = functools.partial(
        pl.kernel,
        mesh=_mesh(),
        compiler_params=pltpu.CompilerParams(use_tc_tiling_on_sc=False),
        out_type=[
            jax.ShapeDtypeStruct((2 * N_NODES, 32), jnp.float32),
            jax.ShapeDtypeStruct((2 * N_NODES, 32), jnp.float32),
        ],
        scratch_types=[
            pltpu.VMEM_SHARED((N_SINK, 32), jnp.float32),
            pltpu.VMEM((EB_ROWS, 128), jnp.int32),
            pltpu.VMEM((EB_ROWS, 128), jnp.int32),
            pltpu.VMEM((256, 32), jnp.float32),
            pltpu.VMEM((256, 32), jnp.float32),
            pltpu.VMEM((NB + 16,), jnp.float32),
            pltpu.VMEM((16,), jnp.float32),
            pltpu.SemaphoreType.DMA,
            pltpu.SemaphoreType.DMA,
            pltpu.SemaphoreType.DMA,
            pltpu.SemaphoreType.DMA,
        ],
    )(functools.partial(_hop_body, n_echunks))
    return kern(yk, hid, srcA, srcB, dst2, dinv, tk)


def kernel(x, edge_index, W1, b1, W2, b2, temp):
    n = x.shape[0]
    e = edge_index.shape[1]
    assert n == N_NODES
    src = edge_index[0]
    dst = edge_index[1]
    e_pad = ((e + EB - 1) // EB) * EB
    if e_pad != e:
        pad = e_pad - e
        src = jnp.concatenate([src, jnp.zeros((pad,), src.dtype)])
        dst = jnp.concatenate(
            [dst, jnp.full((pad,), N_NODES, dst.dtype)])
    src2A = src.reshape(e_pad // EB, EB_ROWS, 128)
    src2B = (src + n).reshape(e_pad // EB, EB_ROWS, 128)
    dst2 = dst.reshape(e_pad // EB, EB_ROWS, 128)

    partials = _deg_call(dst2)
    deg = partials[:n] + partials[n:] + 1.0

    hid, yk, dinv = _mlp_call(x, W1, b1, W2, b2, deg.reshape(n, 1),
                              temp[0].reshape(1, 1))
    hid = hid.reshape(2 * n, 32)
    yk = yk.reshape(2 * n, 32)
    dinv = dinv.reshape(n)

    k_hops = temp.shape[0] - 1
    for k in range(k_hops):
        tk = jnp.broadcast_to(temp[k + 1], (16,))
        yk, hid = _hop_call(yk, hid, src2A, src2B, dst2, dinv, tk)

    hid = hid.reshape(2, n, 32)
    return jnp.concatenate([hid[0], hid[1]], axis=1)
